# Initial kernel scaffold; baseline (speedup 1.0000x reference)
import jax
import jax.numpy as jnp
from jax.experimental import pallas as pl


def kernel(x):
    raise NotImplementedError("write your pallas kernel here")


# speedup vs baseline: 1.0889x; 1.0889x over previous
import jax
import jax.numpy as jnp
from jax import lax
from jax.experimental import pallas as pl

MAX_SIZE = 1000
ROWS_PER_BLOCK = 256


def _onehot_block(x_ref, o_ref):
    xv = x_ref[0, 0, :]
    vocab = lax.broadcasted_iota(jnp.int32, (ROWS_PER_BLOCK, MAX_SIZE), 1)
    o_ref[...] = (xv[:, None] == vocab).astype(jnp.float32)


def kernel(x):
    B, F = x.shape
    n = B * F
    nblocks = n // ROWS_PER_BLOCK
    x3 = x.reshape(nblocks, 1, ROWS_PER_BLOCK)
    out = pl.pallas_call(
        _onehot_block,
        grid=(nblocks,),
        in_specs=[pl.BlockSpec((1, 1, ROWS_PER_BLOCK), lambda i: (i, 0, 0))],
        out_specs=pl.BlockSpec((ROWS_PER_BLOCK, MAX_SIZE), lambda i: (i, 0)),
        out_shape=jax.ShapeDtypeStruct((n, MAX_SIZE), jnp.float32),
    )(x3)
    return out.reshape(B, F, MAX_SIZE)


# speedup vs baseline: 1.7501x; 1.6072x over previous
import jax
import jax.numpy as jnp
from jax import lax
from jax.experimental import pallas as pl

MAX_SIZE = 1000
ROWS_PER_BLOCK = 64


def _onehot_block(x_ref, o_ref):
    xv = x_ref[...]
    vocab = lax.broadcasted_iota(
        jnp.int32, (ROWS_PER_BLOCK, xv.shape[1], MAX_SIZE), 2
    )
    o_ref[...] = (xv[:, :, None] == vocab).astype(jnp.float32)


def kernel(x):
    B, F = x.shape
    nblocks = B // ROWS_PER_BLOCK
    return pl.pallas_call(
        _onehot_block,
        grid=(nblocks,),
        in_specs=[pl.BlockSpec((ROWS_PER_BLOCK, F), lambda i: (i, 0))],
        out_specs=pl.BlockSpec((ROWS_PER_BLOCK, F, MAX_SIZE), lambda i: (i, 0, 0)),
        out_shape=jax.ShapeDtypeStruct((B, F, MAX_SIZE), jnp.float32),
    )(x)
